# baseline (speedup 1.0000x reference)
import functools

import jax
import jax.numpy as jnp
from jax import lax
from jax.experimental import pallas as pl
from jax.experimental.pallas import tpu as pltpu
from jax.experimental.pallas import tpu_sc as plsc

K = 1024
D = 64
BETA = 0.25
N = 32 * 32 * 32
ROWS = 1024
STEPS = N // ROWS
NGRP = K // 128

SC_CORES = 2
SC_SUBCORES = 16
SC_WORKERS = SC_CORES * SC_SUBCORES
ROWS_PER_W = N // SC_WORKERS


def _vq_tc_kernel(zt_ref, emb_ref, embm2_ref, idx_ref, scal_ref, mind_acc):
    b = pl.program_id(0)

    @pl.when(b == 0)
    def _init():
        mind_acc[...] = jnp.zeros_like(mind_acc)

    zt = zt_ref[0]
    emb = emb_ref[...]

    z2 = jnp.sum(zt * zt, axis=0, keepdims=True)
    sqe = emb * emb
    e2b = jax.lax.dot_general(sqe, jnp.ones((D, 128), jnp.float32),
                              (((1,), (0,)), ((), ())),
                              preferred_element_type=jnp.float32)
    e2col = e2b[:, 0:1]
    cm = jax.lax.dot_general(embm2_ref[...], zt, (((1,), (0,)), ((), ())),
                             preferred_element_type=jnp.float32)

    minval = (e2col[0:8] + z2) + cm[0:8, :]
    firstr = jnp.zeros((8, ROWS), jnp.int32)
    for r in range(1, K // 8):
        dg = (e2col[r * 8:(r + 1) * 8] + z2) + cm[r * 8:(r + 1) * 8, :]
        upd = dg < minval
        minval = jnp.where(upd, dg, minval)
        firstr = jnp.where(upd, r, firstr)

    colmin = jnp.min(minval, axis=0, keepdims=True)
    sio = jax.lax.broadcasted_iota(jnp.int32, (8, ROWS), 0)
    key = jnp.where(minval == colmin, firstr * 8 + sio, K)
    idx = jnp.min(key, axis=0, keepdims=True)
    idx_ref[0, 0] = idx[0]
    mind_acc[...] = mind_acc[...] + colmin

    @pl.when(b == STEPS - 1)
    def _finish():
        avg_dist2 = jnp.sum(mind_acc[...]) / jnp.float32(N)
        lane8 = jax.lax.broadcasted_iota(jnp.int32, (1, 8), 1)
        scal_ref[...] = jnp.where(lane8 == 0, (1.0 + BETA) * avg_dist2,
                        jnp.where(lane8 == 4, avg_dist2, 0.0))


def _sc_gather_body(emb_hbm, idx_hbm, zq_hbm, hist_hbm, idx_v, rows_v,
                    hist_v, sem):
    wid = lax.axis_index("s") * SC_CORES + lax.axis_index("c")
    base = wid * ROWS_PER_W
    pltpu.sync_copy(idx_hbm.at[pl.ds(base, ROWS_PER_W)], idx_v)
    pltpu.async_copy(emb_hbm.at[idx_v], rows_v, sem).wait()
    pltpu.sync_copy(rows_v, zq_hbm.at[pl.ds(base, ROWS_PER_W)])

    def zloop(j, _):
        hist_v[pl.ds(j * 16, 16)] = jnp.zeros((16,), jnp.float32)
        return 0

    lax.fori_loop(0, K // 16, zloop, 0)
    ones = jnp.ones((16,), jnp.float32)

    def hloop(j, _):
        ids = idx_v[pl.ds(j * 16, 16)]
        plsc.addupdate_scatter(hist_v, [ids], ones)
        return 0

    lax.fori_loop(0, ROWS_PER_W // 16, hloop, 0)
    pltpu.sync_copy(hist_v, hist_hbm.at[wid])


_sc_gather = pl.kernel(
    _sc_gather_body,
    out_type=(jax.ShapeDtypeStruct((N, D), jnp.float32),
              jax.ShapeDtypeStruct((SC_WORKERS, K), jnp.float32)),
    mesh=plsc.VectorSubcoreMesh(core_axis_name="c", subcore_axis_name="s"),
    scratch_types=[
        pltpu.VMEM((ROWS_PER_W,), jnp.int32),
        pltpu.VMEM((ROWS_PER_W, D), jnp.float32),
        pltpu.VMEM((K,), jnp.float32),
        pltpu.SemaphoreType.DMA,
    ],
    compiler_params=pltpu.CompilerParams(use_tc_tiling_on_sc=False,
                                         needs_layout_passes=False),
)


def _tf_kernel(zq_ref, hist_ref, scala_ref, zqt_ref, out_ref):
    b = pl.program_id(0)
    zqt_ref[0] = jnp.transpose(zq_ref[0], (1, 0))

    @pl.when(b == 0)
    def _fin():
        counts = jnp.sum(hist_ref[...], axis=0, keepdims=True)
        total = jnp.float32(N)
        probs = counts / total
        plogp = jnp.where(probs > 0.0, probs * jnp.log(
            jnp.where(probs > 0.0, probs, 1.0)), 0.0)
        h_ent = -jnp.sum(plogp)
        perplexity = jnp.exp(h_ent)
        codes_used = jnp.sum((counts > 0.0).astype(jnp.float32))
        lane8 = jax.lax.broadcasted_iota(jnp.int32, (1, 8), 1)
        out_ref[...] = jnp.where(lane8 == 1, perplexity,
                       jnp.where(lane8 == 2, codes_used,
                       jnp.where(lane8 == 3, codes_used / jnp.float32(K),
                                 scala_ref[...])))


@jax.jit
def _vq(zt, emb, embm2):
    grid = (STEPS,)
    idx_out, scal_a = pl.pallas_call(
        _vq_tc_kernel,
        grid=grid,
        in_specs=[
            pl.BlockSpec((1, D, ROWS), lambda b: (b, 0, 0)),
            pl.BlockSpec((K, D), lambda b: (0, 0)),
            pl.BlockSpec((K, D), lambda b: (0, 0)),
        ],
        out_specs=[
            pl.BlockSpec((1, 1, ROWS), lambda b: (b, 0, 0)),
            pl.BlockSpec((1, 8), lambda b: (0, 0)),
        ],
        out_shape=[
            jax.ShapeDtypeStruct((STEPS, 1, ROWS), jnp.int32),
            jax.ShapeDtypeStruct((1, 8), jnp.float32),
        ],
        scratch_shapes=[
            pltpu.VMEM((1, ROWS), jnp.float32),
        ],
    )(zt, emb, embm2)
    zq_rows, hist = _sc_gather(emb, idx_out.reshape(N))
    zqt_out, scal = pl.pallas_call(
        _tf_kernel,
        grid=(STEPS,),
        in_specs=[
            pl.BlockSpec((1, ROWS, D), lambda b: (b, 0, 0)),
            pl.BlockSpec((SC_WORKERS, K), lambda b: (0, 0)),
            pl.BlockSpec((1, 8), lambda b: (0, 0)),
        ],
        out_specs=[
            pl.BlockSpec((1, D, ROWS), lambda b: (b, 0, 0)),
            pl.BlockSpec((1, 8), lambda b: (0, 0)),
        ],
        out_shape=[
            jax.ShapeDtypeStruct((STEPS, D, ROWS), jnp.float32),
            jax.ShapeDtypeStruct((1, 8), jnp.float32),
        ],
    )(zq_rows.reshape(STEPS, ROWS, D), hist, scal_a)
    return idx_out, zqt_out, scal


def kernel(z_e, emb):
    B, Dd, H, W = z_e.shape
    zt = z_e.reshape(B, Dd, H * W)
    idx_out, zqt_out, scal = _vq(zt, emb, -2.0 * emb)
    indices = idx_out.reshape(B, H, W)
    z_q_st = zqt_out.reshape(B, Dd, H, W)
    loss_vq = scal[0, 0]
    perplexity = scal[0, 1]
    codes_used = scal[0, 2].astype(jnp.int32)
    usage_ratio = scal[0, 3]
    avg_dist2 = scal[0, 4]
    return (z_q_st, loss_vq, perplexity, codes_used, usage_ratio,
            avg_dist2, indices)


# speedup vs baseline: 1.3605x; 1.3605x over previous
import functools

import jax
import jax.numpy as jnp
from jax import lax
from jax.experimental import pallas as pl
from jax.experimental.pallas import tpu as pltpu
from jax.experimental.pallas import tpu_sc as plsc

K = 1024
D = 64
BETA = 0.25
N = 32 * 32 * 32
ROWS = 1024
STEPS = N // ROWS
NGRP = K // 128

SC_CORES = 2
SC_SUBCORES = 16
SC_WORKERS = SC_CORES * SC_SUBCORES
ROWS_PER_W = N // SC_WORKERS


def _vq_tc_kernel(zt_ref, emb_ref, embm2_ref, idx_ref, scal_ref, mind_acc):
    b = pl.program_id(0)

    @pl.when(b == 0)
    def _init():
        mind_acc[...] = jnp.zeros_like(mind_acc)

    zt = zt_ref[0]
    emb = emb_ref[...]

    z2 = jnp.sum(zt * zt, axis=0, keepdims=True)
    sqe = emb * emb
    e2b = jax.lax.dot_general(sqe, jnp.ones((D, 128), jnp.float32),
                              (((1,), (0,)), ((), ())),
                              preferred_element_type=jnp.float32)
    e2col = e2b[:, 0:1]
    cm = jax.lax.dot_general(embm2_ref[...], zt, (((1,), (0,)), ((), ())),
                             preferred_element_type=jnp.float32)

    minval = (e2col[0:8] + z2) + cm[0:8, :]
    firstr = jnp.zeros((8, ROWS), jnp.int32)
    for r in range(1, K // 8):
        dg = (e2col[r * 8:(r + 1) * 8] + z2) + cm[r * 8:(r + 1) * 8, :]
        upd = dg < minval
        minval = jnp.where(upd, dg, minval)
        firstr = jnp.where(upd, r, firstr)

    colmin = jnp.min(minval, axis=0, keepdims=True)
    sio = jax.lax.broadcasted_iota(jnp.int32, (8, ROWS), 0)
    key = jnp.where(minval == colmin, firstr * 8 + sio, K)
    idx = jnp.min(key, axis=0, keepdims=True)
    idx_ref[0, 0] = idx[0]
    mind_acc[...] = mind_acc[...] + colmin

    @pl.when(b == STEPS - 1)
    def _finish():
        avg_dist2 = jnp.sum(mind_acc[...]) / jnp.float32(N)
        lane8 = jax.lax.broadcasted_iota(jnp.int32, (1, 8), 1)
        scal_ref[...] = jnp.where(lane8 == 0, (1.0 + BETA) * avg_dist2,
                        jnp.where(lane8 == 4, avg_dist2, 0.0))


def _sc_gather_body(emb_hbm, idx_hbm, zq_hbm, hist_hbm, idx_v, rows_v,
                    hist_v, sem):
    wid = lax.axis_index("s") * SC_CORES + lax.axis_index("c")
    base = wid * ROWS_PER_W
    pltpu.sync_copy(idx_hbm.at[pl.ds(base, ROWS_PER_W)], idx_v)
    gather = pltpu.async_copy(emb_hbm.at[idx_v], rows_v, sem)

    def zloop(j, _):
        hist_v[pl.ds(j * 16, 16)] = jnp.zeros((16,), jnp.float32)
        return 0

    lax.fori_loop(0, K // 16, zloop, 0)
    ones = jnp.ones((16,), jnp.float32)

    def hloop(j, _):
        ids = idx_v[pl.ds(j * 16, 16)]
        plsc.addupdate_scatter(hist_v, [ids], ones)
        return 0

    lax.fori_loop(0, ROWS_PER_W // 16, hloop, 0)
    pltpu.sync_copy(hist_v, hist_hbm.at[wid])
    gather.wait()
    pltpu.sync_copy(rows_v, zq_hbm.at[pl.ds(base, ROWS_PER_W)])


_sc_gather = pl.kernel(
    _sc_gather_body,
    out_type=(jax.ShapeDtypeStruct((N, D), jnp.float32),
              jax.ShapeDtypeStruct((SC_WORKERS, K), jnp.float32)),
    mesh=plsc.VectorSubcoreMesh(core_axis_name="c", subcore_axis_name="s"),
    scratch_types=[
        pltpu.VMEM((ROWS_PER_W,), jnp.int32),
        pltpu.VMEM((ROWS_PER_W, D), jnp.float32),
        pltpu.VMEM((K,), jnp.float32),
        pltpu.SemaphoreType.DMA,
    ],
    compiler_params=pltpu.CompilerParams(use_tc_tiling_on_sc=False,
                                         needs_layout_passes=False),
)


def _fin_kernel(hist_ref, scala_ref, out_ref):
    counts = jnp.sum(hist_ref[...], axis=0, keepdims=True)
    total = jnp.float32(N)
    probs = counts / total
    plogp = jnp.where(probs > 0.0, probs * jnp.log(
        jnp.where(probs > 0.0, probs, 1.0)), 0.0)
    h_ent = -jnp.sum(plogp)
    perplexity = jnp.exp(h_ent)
    codes_used = jnp.sum((counts > 0.0).astype(jnp.float32))
    lane8 = jax.lax.broadcasted_iota(jnp.int32, (1, 8), 1)
    out_ref[...] = jnp.where(lane8 == 1, perplexity,
                   jnp.where(lane8 == 2, codes_used,
                   jnp.where(lane8 == 3, codes_used / jnp.float32(K),
                             scala_ref[...])))


@jax.jit
def _vq(zt, emb, embm2):
    grid = (STEPS,)
    idx_out, scal_a = pl.pallas_call(
        _vq_tc_kernel,
        grid=grid,
        in_specs=[
            pl.BlockSpec((1, D, ROWS), lambda b: (b, 0, 0)),
            pl.BlockSpec((K, D), lambda b: (0, 0)),
            pl.BlockSpec((K, D), lambda b: (0, 0)),
        ],
        out_specs=[
            pl.BlockSpec((1, 1, ROWS), lambda b: (b, 0, 0)),
            pl.BlockSpec((1, 8), lambda b: (0, 0)),
        ],
        out_shape=[
            jax.ShapeDtypeStruct((STEPS, 1, ROWS), jnp.int32),
            jax.ShapeDtypeStruct((1, 8), jnp.float32),
        ],
        scratch_shapes=[
            pltpu.VMEM((1, ROWS), jnp.float32),
        ],
    )(zt, emb, embm2)
    zq_rows, hist = _sc_gather(emb, idx_out.reshape(N))
    scal = pl.pallas_call(
        _fin_kernel,
        out_shape=jax.ShapeDtypeStruct((1, 8), jnp.float32),
    )(hist, scal_a)
    return idx_out, zq_rows, scal


def kernel(z_e, emb):
    B, Dd, H, W = z_e.shape
    zt = z_e.reshape(B, Dd, H * W)
    idx_out, zq_rows, scal = _vq(zt, emb, -2.0 * emb)
    indices = idx_out.reshape(B, H, W)
    z_q_st = zq_rows.reshape(B, H, W, Dd).transpose(0, 3, 1, 2)
    loss_vq = scal[0, 0]
    perplexity = scal[0, 1]
    codes_used = scal[0, 2].astype(jnp.int32)
    usage_ratio = scal[0, 3]
    avg_dist2 = scal[0, 4]
    return (z_q_st, loss_vq, perplexity, codes_used, usage_ratio,
            avg_dist2, indices)
